# baseline (speedup 1.0000x reference)
import functools

import jax
import jax.numpy as jnp
from jax import lax
from jax.experimental import pallas as pl
from jax.experimental.pallas import tpu as pltpu
from jax.experimental.pallas import tpu_sc as plsc

_NUM_CORES = 2
_NUM_SUBCORES = 16
_NUM_WORKERS = _NUM_CORES * _NUM_SUBCORES

_LANES = 128
_D = 64
_TILE = 1024


def _fill_perm(perm):
    iota = lax.iota(jnp.int32, 16)
    for dd in range(16):
        col = lax.rem(iota + dd, 16)
        perm[0, dd, :] = col
        perm[1, dd, :] = col * _LANES + iota
        perm[2, dd, :] = iota * _D + col


def _transpose_block(rows, outb, perm):
    iota = lax.iota(jnp.int32, 16)

    def d_body(dd, carry):
        pcol = perm[0, dd, :]
        pdst = perm[1, dd, :]
        for t0 in range(0, _LANES, 16):
            ridx = t0 + iota
            for c0 in range(0, _D, 16):
                vals = plsc.load_gather(rows, [ridx, c0 + pcol])
                plsc.store_scatter(outb, [pdst + (c0 * _LANES + t0)], vals)
        return carry

    lax.fori_loop(0, 16, d_body, 0)


def _transpose_unit(tin, lout, perm):
    iota = lax.iota(jnp.int32, 16)

    def d_body(dd, carry):
        pcol = perm[0, dd, :]
        pdst = perm[2, dd, :]
        for c0 in range(0, _D, 16):
            rsrc = c0 + pcol
            for t0 in range(0, _LANES, 16):
                vals = plsc.load_gather(tin, [rsrc, t0 + iota])
                plsc.store_scatter(lout, [pdst + (t0 * _D + c0)], vals)
        return carry

    lax.fori_loop(0, 16, d_body, 0)


_A_FULL = 7808
_A_UNITS = _A_FULL // _NUM_WORKERS
_A_EXTRA = 5


def _relayout_body(wt_hbm, out_hbm, tin_a, tin_b, lout_a, lout_b, perm,
                   gi_a, gi_b, wo_a, wo_b):
    wid = lax.axis_index("s") * _NUM_CORES + lax.axis_index("c")
    base = wid * _A_UNITS
    _fill_perm(perm)

    def fire_in(j, tin, sem):
        for k in range(_D // 8):
            pltpu.async_copy(
                wt_hbm.at[pl.ds(8 * k, 8), pl.ds(j * _LANES, _LANES)],
                tin.at[pl.ds(8 * k, 8), :], sem)

    def wait_in(tin, sem):
        for k in range(_D // 8):
            pltpu.make_async_copy(
                wt_hbm.at[pl.ds(0, 8), pl.ds(0, _LANES)],
                tin.at[pl.ds(8 * k, 8), :], sem).wait()

    def fire_out(j, lout, sem):
        pltpu.async_copy(lout, out_hbm.at[pl.ds(j * _LANES * _D,
                                                _LANES * _D)], sem)

    def wait_out(lout, sem):
        pltpu.make_async_copy(lout, out_hbm.at[pl.ds(0, _LANES * _D)],
                              sem).wait()

    fire_in(base, tin_a, gi_a)
    wait_in(tin_a, gi_a)
    fire_in(base + 1, tin_b, gi_b)
    _transpose_unit(tin_a, lout_a, perm)
    fire_out(base, lout_a, wo_a)
    wait_in(tin_b, gi_b)
    fire_in(base + 2, tin_a, gi_a)
    _transpose_unit(tin_b, lout_b, perm)
    fire_out(base + 1, lout_b, wo_b)

    def body(i, carry):
        ue = base + 2 * i
        wait_in(tin_a, gi_a)
        fire_in(ue + 1, tin_b, gi_b)
        wait_out(lout_a, wo_a)
        _transpose_unit(tin_a, lout_a, perm)
        fire_out(ue, lout_a, wo_a)
        wait_in(tin_b, gi_b)
        fire_in(ue + 2, tin_a, gi_a)
        wait_out(lout_b, wo_b)
        _transpose_unit(tin_b, lout_b, perm)
        fire_out(ue + 1, lout_b, wo_b)
        return carry

    lax.fori_loop(1, _A_UNITS // 2 - 1, body, 0)

    ue = base + _A_UNITS - 2
    wait_in(tin_a, gi_a)
    fire_in(ue + 1, tin_b, gi_b)
    wait_out(lout_a, wo_a)
    _transpose_unit(tin_a, lout_a, perm)
    fire_out(ue, lout_a, wo_a)
    wait_in(tin_b, gi_b)
    wait_out(lout_b, wo_b)
    _transpose_unit(tin_b, lout_b, perm)
    fire_out(ue + 1, lout_b, wo_b)
    wait_out(lout_a, wo_a)
    wait_out(lout_b, wo_b)

    @pl.when(wid < _A_EXTRA)
    def _extra():
        j = _A_FULL + wid
        for half in range(2):
            for k in range(_D // 8):
                pltpu.async_copy(
                    wt_hbm.at[pl.ds(8 * k, 8),
                              pl.ds(j * _LANES + half * _D, _D)],
                    tin_a.at[pl.ds(8 * k, 8), pl.ds(half * _D, _D)], gi_a)
            for k in range(_D // 8):
                pltpu.make_async_copy(
                    wt_hbm.at[pl.ds(0, 8), pl.ds(0, _D)],
                    tin_a.at[pl.ds(8 * k, 8), pl.ds(half * _D, _D)],
                    gi_a).wait()
        _transpose_unit(tin_a, lout_a, perm)
        fire_out(j, lout_a, wo_a)
        wait_out(lout_a, wo_a)


def _emb_body(blocks_per_w, idx_hbm, table_hbm, out_hbm,
              idx_v, rows_a, rows_b, out_a, out_b, perm,
              gs_a, gs_b, ws_a, ws_b):
    wid = lax.axis_index("s") * _NUM_CORES + lax.axis_index("c")
    jbase = wid * blocks_per_w
    npairs = blocks_per_w // 2

    def fire_g(j, rows, sem):
        pltpu.async_copy(table_hbm.at[idx_v.at[j]], rows, sem)

    def wait_g(rows, sem):
        pltpu.make_async_copy(table_hbm.at[idx_v.at[0]], rows, sem).wait()

    def fire_w(outb, j, sem):
        blk = jbase + j
        s_i = blk // _LANES
        tb = blk - s_i * _LANES
        for tc in range(_D // 8):
            off = ((s_i * (_D // 8) + tc) * _LANES + tb) * _TILE
            pltpu.async_copy(outb.at[pl.ds(tc * _TILE, _TILE)],
                             out_hbm.at[pl.ds(off, _TILE)], sem)

    def wait_w(outb, sem):
        for tc in range(_D // 8):
            pltpu.make_async_copy(outb.at[pl.ds(tc * _TILE, _TILE)],
                                  out_hbm.at[pl.ds(0, _TILE)], sem).wait()

    _fill_perm(perm)

    pltpu.sync_copy(idx_hbm.at[pl.ds(jbase, blocks_per_w)], idx_v)

    fire_g(0, rows_a, gs_a)
    wait_g(rows_a, gs_a)
    fire_g(1, rows_b, gs_b)
    _transpose_block(rows_a, out_a, perm)
    fire_w(out_a, 0, ws_a)
    wait_g(rows_b, gs_b)
    fire_g(2, rows_a, gs_a)
    _transpose_block(rows_b, out_b, perm)
    fire_w(out_b, 1, ws_b)

    def body(i, carry):
        je = 2 * i
        wait_g(rows_a, gs_a)
        fire_g(je + 1, rows_b, gs_b)
        wait_w(out_a, ws_a)
        _transpose_block(rows_a, out_a, perm)
        fire_w(out_a, je, ws_a)
        wait_g(rows_b, gs_b)
        fire_g(je + 2, rows_a, gs_a)
        wait_w(out_b, ws_b)
        _transpose_block(rows_b, out_b, perm)
        fire_w(out_b, je + 1, ws_b)
        return carry

    lax.fori_loop(1, npairs - 1, body, 0)

    je = 2 * npairs - 2
    wait_g(rows_a, gs_a)
    fire_g(je + 1, rows_b, gs_b)
    wait_w(out_a, ws_a)
    _transpose_block(rows_a, out_a, perm)
    fire_w(out_a, je, ws_a)
    wait_g(rows_b, gs_b)
    wait_w(out_b, ws_b)
    _transpose_block(rows_b, out_b, perm)
    fire_w(out_b, je + 1, ws_b)
    wait_w(out_a, ws_a)
    wait_w(out_b, ws_b)


def kernel(token_ids, weight):
    b, s = token_ids.shape
    v, d = weight.shape
    n = b * s
    n_blocks = n // _LANES
    blocks_per_w = n_blocks // _NUM_WORKERS

    tidx = token_ids.T.reshape(n_blocks, _LANES).astype(jnp.int32)

    mesh = plsc.VectorSubcoreMesh(
        core_axis_name="c", subcore_axis_name="s",
        num_cores=_NUM_CORES, num_subcores=_NUM_SUBCORES)

    v_pad = 7813 * _LANES
    relayout = functools.partial(
        pl.kernel,
        out_type=jax.ShapeDtypeStruct((v_pad * d,), jnp.float32),
        mesh=mesh,
        compiler_params=pltpu.CompilerParams(needs_layout_passes=False),
        scratch_types=[
            pltpu.VMEM((d, _LANES), jnp.float32),
            pltpu.VMEM((d, _LANES), jnp.float32),
            pltpu.VMEM((d * _LANES,), jnp.float32),
            pltpu.VMEM((d * _LANES,), jnp.float32),
            pltpu.VMEM((3, 16, 16), jnp.int32),
            pltpu.SemaphoreType.DMA,
            pltpu.SemaphoreType.DMA,
            pltpu.SemaphoreType.DMA,
            pltpu.SemaphoreType.DMA,
        ],
    )(_relayout_body)

    tbl = relayout(weight.T).reshape(v_pad, d)

    emb = functools.partial(
        pl.kernel,
        out_type=jax.ShapeDtypeStruct((n * d,), jnp.float32),
        mesh=mesh,
        compiler_params=pltpu.CompilerParams(use_tc_tiling_on_sc=False,
                                             needs_layout_passes=False),
        scratch_types=[
            pltpu.VMEM((blocks_per_w, _LANES), jnp.int32),
            pltpu.VMEM((_LANES, d), jnp.float32),
            pltpu.VMEM((_LANES, d), jnp.float32),
            pltpu.VMEM((d * _LANES,), jnp.float32),
            pltpu.VMEM((d * _LANES,), jnp.float32),
            pltpu.VMEM((3, 16, 16), jnp.int32),
            pltpu.SemaphoreType.DMA,
            pltpu.SemaphoreType.DMA,
            pltpu.SemaphoreType.DMA,
            pltpu.SemaphoreType.DMA,
        ],
    )(functools.partial(_emb_body, blocks_per_w))

    out_flat = emb(tidx, tbl)
    out5 = out_flat.reshape(s, d // 8, b // _LANES, 8, _LANES)
    return out5.transpose(2, 4, 0, 1, 3).reshape(b, s, d)


# speedup vs baseline: 1.0416x; 1.0416x over previous
import functools

import jax
import jax.numpy as jnp
from jax import lax
from jax.experimental import pallas as pl
from jax.experimental.pallas import tpu as pltpu
from jax.experimental.pallas import tpu_sc as plsc

_NUM_CORES = 2
_NUM_SUBCORES = 16
_NUM_WORKERS = _NUM_CORES * _NUM_SUBCORES

_LANES = 128
_D = 64
_TILE = 1024


def _fill_perm(perm):
    iota = lax.iota(jnp.int32, 16)
    for dd in range(16):
        col = lax.rem(iota + dd, 16)
        perm[0, dd, :] = col
        perm[1, dd, :] = col * _LANES + iota
        perm[2, dd, :] = iota * _D + col
        perm[3, dd, :] = lax.shift_right_logical(col, 3)
        perm[4, dd, :] = lax.bitwise_and(col, 7)


def _transpose_block(rows, outb, perm):
    iota = lax.iota(jnp.int32, 16)

    def d_body(dd, carry):
        pcol = perm[0, dd, :]
        ptc = perm[3, dd, :]
        psub = perm[4, dd, :]
        for t0 in range(0, _LANES, 16):
            ridx = t0 + iota
            for c0 in range(0, _D, 16):
                vals = plsc.load_gather(rows, [ridx, c0 + pcol])
                plsc.store_scatter(outb, [ptc + c0 // 8, psub, ridx], vals)
        return carry

    lax.fori_loop(0, 16, d_body, 0)


def _transpose_unit(tin, lout, perm):
    iota = lax.iota(jnp.int32, 16)

    def d_body(dd, carry):
        pcol = perm[0, dd, :]
        pdst = perm[2, dd, :]
        for c0 in range(0, _D, 16):
            rsrc = c0 + pcol
            for t0 in range(0, _LANES, 16):
                vals = plsc.load_gather(tin, [rsrc, t0 + iota])
                plsc.store_scatter(lout, [pdst + (t0 * _D + c0)], vals)
        return carry

    lax.fori_loop(0, 16, d_body, 0)


_A_FULL = 7808
_A_UNITS = _A_FULL // _NUM_WORKERS
_A_EXTRA = 5


def _relayout_body(wt_hbm, out_hbm, tin_a, tin_b, lout_a, lout_b, perm,
                   gi_a, gi_b, wo_a, wo_b):
    wid = lax.axis_index("s") * _NUM_CORES + lax.axis_index("c")
    base = wid * _A_UNITS
    _fill_perm(perm)

    def fire_in(j, tin, sem):
        pltpu.async_copy(wt_hbm.at[:, pl.ds(j * _LANES, _LANES)], tin, sem)

    def wait_in(tin, sem):
        pltpu.make_async_copy(wt_hbm.at[:, pl.ds(0, _LANES)], tin,
                              sem).wait()

    def fire_out(j, lout, sem):
        pltpu.async_copy(lout, out_hbm.at[pl.ds(j * _LANES * _D,
                                                _LANES * _D)], sem)

    def wait_out(lout, sem):
        pltpu.make_async_copy(lout, out_hbm.at[pl.ds(0, _LANES * _D)],
                              sem).wait()

    fire_in(base, tin_a, gi_a)
    wait_in(tin_a, gi_a)
    fire_in(base + 1, tin_b, gi_b)
    _transpose_unit(tin_a, lout_a, perm)
    fire_out(base, lout_a, wo_a)
    wait_in(tin_b, gi_b)
    fire_in(base + 2, tin_a, gi_a)
    _transpose_unit(tin_b, lout_b, perm)
    fire_out(base + 1, lout_b, wo_b)

    def body(i, carry):
        ue = base + 2 * i
        wait_in(tin_a, gi_a)
        fire_in(ue + 1, tin_b, gi_b)
        wait_out(lout_a, wo_a)
        _transpose_unit(tin_a, lout_a, perm)
        fire_out(ue, lout_a, wo_a)
        wait_in(tin_b, gi_b)
        fire_in(ue + 2, tin_a, gi_a)
        wait_out(lout_b, wo_b)
        _transpose_unit(tin_b, lout_b, perm)
        fire_out(ue + 1, lout_b, wo_b)
        return carry

    lax.fori_loop(1, _A_UNITS // 2 - 1, body, 0)

    ue = base + _A_UNITS - 2
    wait_in(tin_a, gi_a)
    fire_in(ue + 1, tin_b, gi_b)
    wait_out(lout_a, wo_a)
    _transpose_unit(tin_a, lout_a, perm)
    fire_out(ue, lout_a, wo_a)
    wait_in(tin_b, gi_b)
    wait_out(lout_b, wo_b)
    _transpose_unit(tin_b, lout_b, perm)
    fire_out(ue + 1, lout_b, wo_b)
    wait_out(lout_a, wo_a)
    wait_out(lout_b, wo_b)

    @pl.when(wid < _A_EXTRA)
    def _extra():
        j = _A_FULL + wid
        for half in range(2):
            for k in range(_D // 8):
                pltpu.async_copy(
                    wt_hbm.at[pl.ds(8 * k, 8),
                              pl.ds(j * _LANES + half * _D, _D)],
                    tin_a.at[pl.ds(8 * k, 8), pl.ds(half * _D, _D)], gi_a)
            for k in range(_D // 8):
                pltpu.make_async_copy(
                    wt_hbm.at[pl.ds(0, 8), pl.ds(0, _D)],
                    tin_a.at[pl.ds(8 * k, 8), pl.ds(half * _D, _D)],
                    gi_a).wait()
        _transpose_unit(tin_a, lout_a, perm)
        fire_out(j, lout_a, wo_a)
        wait_out(lout_a, wo_a)


def _emb_body(blocks_per_w, idx_hbm, table_hbm, out_hbm,
              idx_v, rows_a, rows_b, out_a, out_b, perm,
              gs_a, gs_b, ws_a, ws_b):
    wid = lax.axis_index("s") * _NUM_CORES + lax.axis_index("c")
    jbase = wid * blocks_per_w
    npairs = blocks_per_w // 2

    def fire_g(j, rows, sem):
        pltpu.async_copy(table_hbm.at[idx_v.at[j]], rows, sem)

    def wait_g(rows, sem):
        pltpu.make_async_copy(table_hbm.at[idx_v.at[0]], rows, sem).wait()

    def fire_w(outb, j, sem):
        blk = jbase + j
        s_i = blk // _LANES
        tb = blk - s_i * _LANES
        pltpu.async_copy(outb, out_hbm.at[s_i, :, tb], sem)

    def wait_w(outb, sem):
        pltpu.make_async_copy(outb, out_hbm.at[0, :, 0], sem).wait()

    _fill_perm(perm)

    pltpu.sync_copy(idx_hbm.at[pl.ds(jbase, blocks_per_w)], idx_v)

    fire_g(0, rows_a, gs_a)
    wait_g(rows_a, gs_a)
    fire_g(1, rows_b, gs_b)
    _transpose_block(rows_a, out_a, perm)
    fire_w(out_a, 0, ws_a)
    wait_g(rows_b, gs_b)
    fire_g(2, rows_a, gs_a)
    _transpose_block(rows_b, out_b, perm)
    fire_w(out_b, 1, ws_b)

    def body(i, carry):
        je = 2 * i
        wait_g(rows_a, gs_a)
        fire_g(je + 1, rows_b, gs_b)
        wait_w(out_a, ws_a)
        _transpose_block(rows_a, out_a, perm)
        fire_w(out_a, je, ws_a)
        wait_g(rows_b, gs_b)
        fire_g(je + 2, rows_a, gs_a)
        wait_w(out_b, ws_b)
        _transpose_block(rows_b, out_b, perm)
        fire_w(out_b, je + 1, ws_b)
        return carry

    lax.fori_loop(1, npairs - 1, body, 0)

    je = 2 * npairs - 2
    wait_g(rows_a, gs_a)
    fire_g(je + 1, rows_b, gs_b)
    wait_w(out_a, ws_a)
    _transpose_block(rows_a, out_a, perm)
    fire_w(out_a, je, ws_a)
    wait_g(rows_b, gs_b)
    wait_w(out_b, ws_b)
    _transpose_block(rows_b, out_b, perm)
    fire_w(out_b, je + 1, ws_b)
    wait_w(out_a, ws_a)
    wait_w(out_b, ws_b)


def kernel(token_ids, weight):
    b, s = token_ids.shape
    v, d = weight.shape
    n = b * s
    n_blocks = n // _LANES
    blocks_per_w = n_blocks // _NUM_WORKERS

    tidx = token_ids.T.reshape(n_blocks, _LANES).astype(jnp.int32)

    mesh = plsc.VectorSubcoreMesh(
        core_axis_name="c", subcore_axis_name="s",
        num_cores=_NUM_CORES, num_subcores=_NUM_SUBCORES)

    v_pad = 7813 * _LANES
    relayout = functools.partial(
        pl.kernel,
        out_type=jax.ShapeDtypeStruct((v_pad * d,), jnp.float32),
        mesh=mesh,
        compiler_params=pltpu.CompilerParams(needs_layout_passes=False),
        scratch_types=[
            pltpu.VMEM((d, _LANES), jnp.float32),
            pltpu.VMEM((d, _LANES), jnp.float32),
            pltpu.VMEM((d * _LANES,), jnp.float32),
            pltpu.VMEM((d * _LANES,), jnp.float32),
            pltpu.VMEM((5, 16, 16), jnp.int32),
            pltpu.SemaphoreType.DMA,
            pltpu.SemaphoreType.DMA,
            pltpu.SemaphoreType.DMA,
            pltpu.SemaphoreType.DMA,
        ],
    )(_relayout_body)

    tbl = relayout(weight.T).reshape(v_pad, d)

    emb = functools.partial(
        pl.kernel,
        out_type=jax.ShapeDtypeStruct((s, d // 8, b // _LANES, 8, _LANES),
                                      jnp.float32),
        mesh=mesh,
        compiler_params=pltpu.CompilerParams(use_tc_tiling_on_sc=False,
                                             needs_layout_passes=False),
        scratch_types=[
            pltpu.VMEM((blocks_per_w, _LANES), jnp.int32),
            pltpu.VMEM((_LANES, d), jnp.float32),
            pltpu.VMEM((_LANES, d), jnp.float32),
            pltpu.VMEM((d // 8, 8, _LANES), jnp.float32),
            pltpu.VMEM((d // 8, 8, _LANES), jnp.float32),
            pltpu.VMEM((5, 16, 16), jnp.int32),
            pltpu.SemaphoreType.DMA,
            pltpu.SemaphoreType.DMA,
            pltpu.SemaphoreType.DMA,
            pltpu.SemaphoreType.DMA,
        ],
    )(functools.partial(_emb_body, blocks_per_w))

    out5 = emb(tidx, tbl)
    return out5.transpose(2, 4, 0, 1, 3).reshape(b, s, d)


# speedup vs baseline: 1.7728x; 1.7020x over previous
import functools

import jax
import jax.numpy as jnp
from jax import lax
from jax.experimental import pallas as pl
from jax.experimental.pallas import tpu as pltpu
from jax.experimental.pallas import tpu_sc as plsc

_NUM_CORES = 2
_NUM_SUBCORES = 16
_NUM_WORKERS = _NUM_CORES * _NUM_SUBCORES

_LANES = 128
_D = 64
_TILE = 1024


def _fill_perm(perm):
    iota = lax.iota(jnp.int32, 16)
    for dd in range(16):
        col = lax.rem(iota + dd, 16)
        perm[0, dd, :] = col
        perm[1, dd, :] = col * _LANES + iota
        perm[2, dd, :] = iota * _D + col
        perm[3, dd, :] = lax.shift_right_logical(col, 3)
        perm[4, dd, :] = lax.bitwise_and(col, 7)


def _transpose_block(rows, outb, perm):
    iota = lax.iota(jnp.int32, 16)

    @functools.partial(plsc.parallel_loop, 0, 16, unroll=2)
    def _(dd):
        pcol = perm[0, dd, :]
        ptc = perm[3, dd, :]
        psub = perm[4, dd, :]
        for t0 in range(0, _LANES, 16):
            ridx = t0 + iota
            for c0 in range(0, _D, 16):
                vals = plsc.load_gather(rows, [ridx, c0 + pcol])
                plsc.store_scatter(outb, [ptc + c0 // 8, psub, ridx], vals)


def _transpose_unit(tin, lout, perm):
    iota = lax.iota(jnp.int32, 16)

    @functools.partial(plsc.parallel_loop, 0, 16, unroll=2)
    def _(dd):
        pcol = perm[0, dd, :]
        pdst = perm[2, dd, :]
        for c0 in range(0, _D, 16):
            rsrc = c0 + pcol
            for t0 in range(0, _LANES, 16):
                vals = plsc.load_gather(tin, [rsrc, t0 + iota])
                plsc.store_scatter(lout, [pdst + (t0 * _D + c0)], vals)


_A_FULL = 7808
_A_UNITS = _A_FULL // _NUM_WORKERS
_A_EXTRA = 5


def _relayout_body(wt_hbm, out_hbm, tin_a, tin_b, lout_a, lout_b, perm,
                   gi_a, gi_b, wo_a, wo_b):
    wid = lax.axis_index("s") * _NUM_CORES + lax.axis_index("c")
    base = wid * _A_UNITS
    _fill_perm(perm)

    def fire_in(j, tin, sem):
        pltpu.async_copy(wt_hbm.at[:, pl.ds(j * _LANES, _LANES)], tin, sem)

    def wait_in(tin, sem):
        pltpu.make_async_copy(wt_hbm.at[:, pl.ds(0, _LANES)], tin,
                              sem).wait()

    def fire_out(j, lout, sem):
        pltpu.async_copy(lout, out_hbm.at[pl.ds(j * _LANES * _D,
                                                _LANES * _D)], sem)

    def wait_out(lout, sem):
        pltpu.make_async_copy(lout, out_hbm.at[pl.ds(0, _LANES * _D)],
                              sem).wait()

    fire_in(base, tin_a, gi_a)
    wait_in(tin_a, gi_a)
    fire_in(base + 1, tin_b, gi_b)
    _transpose_unit(tin_a, lout_a, perm)
    fire_out(base, lout_a, wo_a)
    wait_in(tin_b, gi_b)
    fire_in(base + 2, tin_a, gi_a)
    _transpose_unit(tin_b, lout_b, perm)
    fire_out(base + 1, lout_b, wo_b)

    def body(i, carry):
        ue = base + 2 * i
        wait_in(tin_a, gi_a)
        fire_in(ue + 1, tin_b, gi_b)
        wait_out(lout_a, wo_a)
        _transpose_unit(tin_a, lout_a, perm)
        fire_out(ue, lout_a, wo_a)
        wait_in(tin_b, gi_b)
        fire_in(ue + 2, tin_a, gi_a)
        wait_out(lout_b, wo_b)
        _transpose_unit(tin_b, lout_b, perm)
        fire_out(ue + 1, lout_b, wo_b)
        return carry

    lax.fori_loop(1, _A_UNITS // 2 - 1, body, 0)

    ue = base + _A_UNITS - 2
    wait_in(tin_a, gi_a)
    fire_in(ue + 1, tin_b, gi_b)
    wait_out(lout_a, wo_a)
    _transpose_unit(tin_a, lout_a, perm)
    fire_out(ue, lout_a, wo_a)
    wait_in(tin_b, gi_b)
    wait_out(lout_b, wo_b)
    _transpose_unit(tin_b, lout_b, perm)
    fire_out(ue + 1, lout_b, wo_b)
    wait_out(lout_a, wo_a)
    wait_out(lout_b, wo_b)

    @pl.when(wid < _A_EXTRA)
    def _extra():
        j = _A_FULL + wid
        for half in range(2):
            for k in range(_D // 8):
                pltpu.async_copy(
                    wt_hbm.at[pl.ds(8 * k, 8),
                              pl.ds(j * _LANES + half * _D, _D)],
                    tin_a.at[pl.ds(8 * k, 8), pl.ds(half * _D, _D)], gi_a)
            for k in range(_D // 8):
                pltpu.make_async_copy(
                    wt_hbm.at[pl.ds(0, 8), pl.ds(0, _D)],
                    tin_a.at[pl.ds(8 * k, 8), pl.ds(half * _D, _D)],
                    gi_a).wait()
        _transpose_unit(tin_a, lout_a, perm)
        fire_out(j, lout_a, wo_a)
        wait_out(lout_a, wo_a)


def _emb_body(blocks_per_w, idx_hbm, table_hbm, out_hbm,
              idx_v, rows_a, rows_b, out_a, out_b, perm,
              gs_a, gs_b, ws_a, ws_b):
    wid = lax.axis_index("s") * _NUM_CORES + lax.axis_index("c")
    jbase = wid * blocks_per_w
    npairs = blocks_per_w // 2

    def fire_g(j, rows, sem):
        pltpu.async_copy(table_hbm.at[idx_v.at[j]], rows, sem)

    def wait_g(rows, sem):
        pltpu.make_async_copy(table_hbm.at[idx_v.at[0]], rows, sem).wait()

    def fire_w(outb, j, sem):
        blk = jbase + j
        s_i = blk // _LANES
        tb = blk - s_i * _LANES
        pltpu.async_copy(outb, out_hbm.at[s_i, :, tb], sem)

    def wait_w(outb, sem):
        pltpu.make_async_copy(outb, out_hbm.at[0, :, 0], sem).wait()

    _fill_perm(perm)

    pltpu.sync_copy(idx_hbm.at[pl.ds(jbase, blocks_per_w)], idx_v)

    fire_g(0, rows_a, gs_a)
    wait_g(rows_a, gs_a)
    fire_g(1, rows_b, gs_b)
    _transpose_block(rows_a, out_a, perm)
    fire_w(out_a, 0, ws_a)
    wait_g(rows_b, gs_b)
    fire_g(2, rows_a, gs_a)
    _transpose_block(rows_b, out_b, perm)
    fire_w(out_b, 1, ws_b)

    def body(i, carry):
        je = 2 * i
        wait_g(rows_a, gs_a)
        fire_g(je + 1, rows_b, gs_b)
        wait_w(out_a, ws_a)
        _transpose_block(rows_a, out_a, perm)
        fire_w(out_a, je, ws_a)
        wait_g(rows_b, gs_b)
        fire_g(je + 2, rows_a, gs_a)
        wait_w(out_b, ws_b)
        _transpose_block(rows_b, out_b, perm)
        fire_w(out_b, je + 1, ws_b)
        return carry

    lax.fori_loop(1, npairs - 1, body, 0)

    je = 2 * npairs - 2
    wait_g(rows_a, gs_a)
    fire_g(je + 1, rows_b, gs_b)
    wait_w(out_a, ws_a)
    _transpose_block(rows_a, out_a, perm)
    fire_w(out_a, je, ws_a)
    wait_g(rows_b, gs_b)
    wait_w(out_b, ws_b)
    _transpose_block(rows_b, out_b, perm)
    fire_w(out_b, je + 1, ws_b)
    wait_w(out_a, ws_a)
    wait_w(out_b, ws_b)


def kernel(token_ids, weight):
    b, s = token_ids.shape
    v, d = weight.shape
    n = b * s
    n_blocks = n // _LANES
    blocks_per_w = n_blocks // _NUM_WORKERS

    tidx = token_ids.T.reshape(n_blocks, _LANES).astype(jnp.int32)

    mesh = plsc.VectorSubcoreMesh(
        core_axis_name="c", subcore_axis_name="s",
        num_cores=_NUM_CORES, num_subcores=_NUM_SUBCORES)

    v_pad = 7813 * _LANES
    relayout = functools.partial(
        pl.kernel,
        out_type=jax.ShapeDtypeStruct((v_pad * d,), jnp.float32),
        mesh=mesh,
        compiler_params=pltpu.CompilerParams(needs_layout_passes=False),
        scratch_types=[
            pltpu.VMEM((d, _LANES), jnp.float32),
            pltpu.VMEM((d, _LANES), jnp.float32),
            pltpu.VMEM((d * _LANES,), jnp.float32),
            pltpu.VMEM((d * _LANES,), jnp.float32),
            pltpu.VMEM((5, 16, 16), jnp.int32),
            pltpu.SemaphoreType.DMA,
            pltpu.SemaphoreType.DMA,
            pltpu.SemaphoreType.DMA,
            pltpu.SemaphoreType.DMA,
        ],
    )(_relayout_body)

    tbl = relayout(weight.T).reshape(v_pad, d)

    emb = functools.partial(
        pl.kernel,
        out_type=jax.ShapeDtypeStruct((s, d // 8, b // _LANES, 8, _LANES),
                                      jnp.float32),
        mesh=mesh,
        compiler_params=pltpu.CompilerParams(use_tc_tiling_on_sc=False,
                                             needs_layout_passes=False),
        scratch_types=[
            pltpu.VMEM((blocks_per_w, _LANES), jnp.int32),
            pltpu.VMEM((_LANES, d), jnp.float32),
            pltpu.VMEM((_LANES, d), jnp.float32),
            pltpu.VMEM((d // 8, 8, _LANES), jnp.float32),
            pltpu.VMEM((d // 8, 8, _LANES), jnp.float32),
            pltpu.VMEM((5, 16, 16), jnp.int32),
            pltpu.SemaphoreType.DMA,
            pltpu.SemaphoreType.DMA,
            pltpu.SemaphoreType.DMA,
            pltpu.SemaphoreType.DMA,
        ],
    )(functools.partial(_emb_body, blocks_per_w))

    out5 = emb(tidx, tbl)
    return out5.transpose(2, 4, 0, 1, 3).reshape(b, s, d)
